# Initial kernel scaffold; baseline (speedup 1.0000x reference)
import jax
import jax.numpy as jnp
from jax.experimental import pallas as pl


def kernel(x, token_mask):
    raise NotImplementedError("write your pallas kernel here")


# speedup vs baseline: 3.1443x; 3.1443x over previous
import jax
import jax.numpy as jnp
from jax.experimental import pallas as pl

_K = 64
_ROWS_PER_BLOCK = 8
_MSB = -2147483648


def _topk_mask_kernel(x_ref, o_ref):
    xb = x_ref[...]
    u = jax.lax.bitcast_convert_type(xb, jnp.int32)
    key = u ^ (
        jax.lax.shift_right_arithmetic(u, 31).astype(jnp.int32)
        & 2147483647
    )

    def val_step(i, w):
        bit = jax.lax.shift_left(jnp.int32(1), 31 - i)
        cand_w = w | bit
        cand_v = cand_w ^ _MSB
        cnt = jnp.sum((key >= cand_v).astype(jnp.int32), axis=1, keepdims=True)
        return jnp.where(cnt >= _K, cand_w, w)

    w0 = jnp.zeros((xb.shape[0], 1), dtype=jnp.int32)
    w = jax.lax.fori_loop(0, 32, val_step, w0)
    t = w ^ _MSB

    eq = key == t
    c_gt = jnp.sum((key > t).astype(jnp.int32), axis=1, keepdims=True)
    extras = _K - c_gt

    idx = jax.lax.broadcasted_iota(jnp.int32, xb.shape, 1)

    def idx_step(i, c):
        cand = c | jax.lax.shift_left(jnp.int32(1), 15 - i)
        h = jnp.sum((eq & (idx < cand)).astype(jnp.int32), axis=1, keepdims=True)
        return jnp.where(h <= extras, cand, c)

    c0 = jnp.zeros((xb.shape[0], 1), dtype=jnp.int32)
    cut = jax.lax.fori_loop(0, 16, idx_step, c0)

    mask = (key > t) | (eq & (idx < cut))
    o_ref[...] = jnp.where(mask, jnp.maximum(xb, 0.0), 0.0)


@jax.jit
def kernel(x, token_mask):
    del token_mask
    rows, n = x.shape
    grid = rows // _ROWS_PER_BLOCK
    return pl.pallas_call(
        _topk_mask_kernel,
        grid=(grid,),
        in_specs=[pl.BlockSpec((_ROWS_PER_BLOCK, n), lambda i: (i, 0))],
        out_specs=pl.BlockSpec((_ROWS_PER_BLOCK, n), lambda i: (i, 0)),
        out_shape=jax.ShapeDtypeStruct((rows, n), x.dtype),
    )(x)


# speedup vs baseline: 5.5468x; 1.7641x over previous
import functools

import jax
import jax.numpy as jnp
from jax import lax
from jax.experimental import pallas as pl
from jax.experimental.pallas import tpu as pltpu
from jax.experimental.pallas import tpu_sc as plsc

ROWS = 128
N = 32768
K = 64
L = 16
NVREG = N // L
GROUPS = 64
VPG = NVREG // GROUPS
IMIN = -2147483648
NC = 2
NS = 16
ROWS_PER = ROWS // (NC * NS)


def _sc_body(x_hbm, o_hbm, xv, ck, ci, wv, wi):
    wid = lax.axis_index("s") * NC + lax.axis_index("c")

    def do_row(r, carry):
        row = wid * ROWS_PER + r
        pltpu.sync_copy(x_hbm.at[row], xv)

        def group_body(g, t0):
            def vmax_body(j, acc):
                return jnp.maximum(acc, xv[pl.ds((g * VPG + j) * L, L)])
            acc = lax.fori_loop(1, VPG, vmax_body, xv[pl.ds(g * VPG * L, L)])
            return jnp.minimum(t0, jnp.max(acc))
        t0 = lax.fori_loop(0, GROUPS, group_body, jnp.float32(jnp.inf))

        def compact_body(v, off):
            base = v * L
            x16 = xv[pl.ds(base, L)]
            m = x16 >= t0
            u = plsc.bitcast(x16, jnp.int32)
            k16 = u ^ ((u >> 31) & 0x7FFFFFFF)
            i16 = lax.iota(jnp.int32, L) + base
            plsc.store_compressed(ck.at[pl.ds(off, L)], k16, mask=m)
            plsc.store_compressed(ci.at[pl.ds(off, L)], i16, mask=m)
            xv[pl.ds(base, L)] = jnp.zeros((L,), jnp.float32)
            pc = plsc.all_reduce_population_count(m)
            return off + pc[0]
        n = lax.fori_loop(0, NVREG, compact_body, jnp.int32(0))

        for j in range(4):
            ck[pl.ds(n + j * L, L)] = jnp.full((L,), IMIN, jnp.int32)
        nv = (n + L - 1) // L

        def bit_body(i, w):
            cand_w = w | lax.shift_left(jnp.int32(1), 31 - i)
            cand_v = cand_w ^ IMIN
            def cnt_body(j, acc):
                return acc + jnp.where(ck[pl.ds(j * L, L)] >= cand_v, 1, 0)
            cnt = jnp.sum(lax.fori_loop(0, nv, cnt_body, jnp.zeros((L,), jnp.int32)))
            return jnp.where(cnt >= K, cand_w, w)
        t = lax.fori_loop(0, 32, bit_body, jnp.int32(0)) ^ IMIN

        def cgt_body(j, acc):
            return acc + jnp.where(ck[pl.ds(j * L, L)] > t, 1, 0)
        c_gt = jnp.sum(lax.fori_loop(0, nv, cgt_body, jnp.zeros((L,), jnp.int32)))
        extras = K - c_gt

        def cut_body(i, c):
            cand = c | lax.shift_left(jnp.int32(1), 15 - i)
            def h_body(j, acc):
                k16 = ck[pl.ds(j * L, L)]
                i16 = ci[pl.ds(j * L, L)]
                return acc + jnp.where((k16 == t) & (i16 < cand), 1, 0)
            h = jnp.sum(lax.fori_loop(0, nv, h_body, jnp.zeros((L,), jnp.int32)))
            return jnp.where(h <= extras, cand, c)
        cut = lax.fori_loop(0, 16, cut_body, jnp.int32(0))

        def win_body(j, off2):
            k16 = ck[pl.ds(j * L, L)]
            i16 = ci[pl.ds(j * L, L)]
            m = (k16 > t) | ((k16 == t) & (i16 < cut))
            u = k16 ^ ((k16 >> 31) & 0x7FFFFFFF)
            val = jnp.maximum(plsc.bitcast(u, jnp.float32), 0.0)
            plsc.store_compressed(wv.at[pl.ds(off2, L)], val, mask=m)
            plsc.store_compressed(wi.at[pl.ds(off2, L)], i16, mask=m)
            pc = plsc.all_reduce_population_count(m)
            return off2 + pc[0]
        lax.fori_loop(0, nv, win_body, jnp.int32(0))

        for j in range(4):
            plsc.store_scatter(xv, [wi[pl.ds(j * L, L)]], wv[pl.ds(j * L, L)])
        pltpu.sync_copy(xv, o_hbm.at[row])
        return carry

    lax.fori_loop(0, ROWS_PER, do_row, jnp.int32(0))


def _make(interpret=False):
    return functools.partial(
        pl.kernel,
        out_type=jax.ShapeDtypeStruct((ROWS, N), jnp.float32),
        mesh=plsc.VectorSubcoreMesh(
            core_axis_name="c", subcore_axis_name="s",
            num_cores=NC, num_subcores=NS,
        ),
        scratch_types=[
            pltpu.VMEM((N,), jnp.float32),
            pltpu.VMEM((N + 2 * K,), jnp.int32),
            pltpu.VMEM((N + 2 * K,), jnp.int32),
            pltpu.VMEM((K + L,), jnp.float32),
            pltpu.VMEM((K + L,), jnp.int32),
        ],
        compiler_params=pltpu.CompilerParams(needs_layout_passes=False),
        interpret=interpret,
    )(_sc_body)


_sc_call = _make()


@jax.jit
def kernel(x, token_mask):
    del token_mask
    return _sc_call(x)


if __name__ == "__main__":
    import reference as ref
    f = _make(interpret=True)
    d = ref.setup_inputs(0)
    out = f(d["x"])
    r = ref.reference(d["x"], d["token_mask"])
    print("max_abs_err", float(jnp.max(jnp.abs(out - r))),
          "exact", bool(jnp.all(out == r)))


# speedup vs baseline: 7.3666x; 1.3281x over previous
import functools

import jax
import jax.numpy as jnp
from jax import lax
from jax.experimental import pallas as pl
from jax.experimental.pallas import tpu as pltpu
from jax.experimental.pallas import tpu_sc as plsc

ROWS = 128
N = 32768
K = 64
L = 16
NVREG = N // L
IMIN = -2147483648
NC = 2
NS = 16
ROWS_PER = ROWS // (NC * NS)


def _key(x16):
    u = plsc.bitcast(x16, jnp.int32)
    return u ^ ((u >> 31) & 0x7FFFFFFF)


def _sc_body(x_hbm, o_hbm, xv, ck, ci, wv, wi):
    wid = lax.axis_index("s") * NC + lax.axis_index("c")

    def do_row(r, carry):
        row = wid * ROWS_PER + r
        pltpu.sync_copy(x_hbm.at[row], xv)

        def a_body(i, accs):
            accs = list(accs)
            base = i * 8 * L
            for u in range(8):
                x16 = xv[pl.ds(base + u * L, L)]
                accs[u % 4] = jnp.maximum(accs[u % 4], x16)
            return tuple(accs)
        ninf = jnp.full((L,), -jnp.inf, jnp.float32)
        a0, a1, a2, a3 = lax.fori_loop(
            0, NVREG // 8, a_body, (ninf, ninf, ninf, ninf))
        t0 = jnp.min(jnp.minimum(jnp.minimum(a0, a1), jnp.minimum(a2, a3)))

        def b_body(i, off):
            base = i * 4 * L
            for u in range(4):
                x16 = xv[pl.ds(base + u * L, L)]
                m = x16 >= t0
                k16 = _key(x16)
                i16 = lax.iota(jnp.int32, L) + (base + u * L)
                plsc.store_compressed(ck.at[pl.ds(off, L)], k16, mask=m)
                plsc.store_compressed(ci.at[pl.ds(off, L)], i16, mask=m)
                xv[pl.ds(base + u * L, L)] = jnp.zeros((L,), jnp.float32)
                off = off + plsc.all_reduce_population_count(m)[0]
            return off
        n = lax.fori_loop(0, NVREG // 4, b_body, jnp.int32(0))

        for j in range(4):
            ck[pl.ds(n + j * L, L)] = jnp.full((L,), IMIN, jnp.int32)
        nv2 = (n + 2 * L - 1) // (2 * L)

        def count2(pred):
            def body(j, acc):
                acc = acc + jnp.where(pred(ck[pl.ds(j * 2 * L, L)],
                                           ci[pl.ds(j * 2 * L, L)]), 1, 0)
                return acc + jnp.where(pred(ck[pl.ds(j * 2 * L + L, L)],
                                            ci[pl.ds(j * 2 * L + L, L)]), 1, 0)
            return jnp.sum(lax.fori_loop(0, nv2, body, jnp.zeros((L,), jnp.int32)))

        def bit_body(i, w):
            cand_w = w | lax.shift_left(jnp.int32(1), 31 - i)
            cand_v = cand_w ^ IMIN
            cnt = count2(lambda k16, i16: k16 >= cand_v)
            return jnp.where(cnt >= K, cand_w, w)
        t = lax.fori_loop(0, 32, bit_body, jnp.int32(0)) ^ IMIN

        c_gt = count2(lambda k16, i16: k16 > t)
        extras = K - c_gt

        def cut_body(i, c):
            cand = c | lax.shift_left(jnp.int32(1), 15 - i)
            h = count2(lambda k16, i16: (k16 == t) & (i16 < cand))
            return jnp.where(h <= extras, cand, c)
        cut = lax.fori_loop(0, 16, cut_body, jnp.int32(0))

        def win_body(j, off2):
            for u in range(2):
                k16 = ck[pl.ds(j * 2 * L + u * L, L)]
                i16 = ci[pl.ds(j * 2 * L + u * L, L)]
                m = (k16 > t) | ((k16 == t) & (i16 < cut))
                val = jnp.maximum(
                    plsc.bitcast(k16 ^ ((k16 >> 31) & 0x7FFFFFFF), jnp.float32),
                    0.0)
                plsc.store_compressed(wv.at[pl.ds(off2, L)], val, mask=m)
                plsc.store_compressed(wi.at[pl.ds(off2, L)], i16, mask=m)
                off2 = off2 + plsc.all_reduce_population_count(m)[0]
            return off2
        lax.fori_loop(0, nv2, win_body, jnp.int32(0))

        for j in range(4):
            plsc.store_scatter(xv, [wi[pl.ds(j * L, L)]], wv[pl.ds(j * L, L)])
        pltpu.sync_copy(xv, o_hbm.at[row])
        return carry

    lax.fori_loop(0, ROWS_PER, do_row, jnp.int32(0))


_sc_call = functools.partial(
    pl.kernel,
    out_type=jax.ShapeDtypeStruct((ROWS, N), jnp.float32),
    mesh=plsc.VectorSubcoreMesh(
        core_axis_name="c", subcore_axis_name="s",
        num_cores=NC, num_subcores=NS,
    ),
    scratch_types=[
        pltpu.VMEM((N,), jnp.float32),
        pltpu.VMEM((N + 2 * K,), jnp.int32),
        pltpu.VMEM((N + 2 * K,), jnp.int32),
        pltpu.VMEM((K + L,), jnp.float32),
        pltpu.VMEM((K + L,), jnp.int32),
    ],
    compiler_params=pltpu.CompilerParams(needs_layout_passes=False),
)(_sc_body)


@jax.jit
def kernel(x, token_mask):
    del token_mask
    return _sc_call(x)


# speedup vs baseline: 9.8208x; 1.3332x over previous
import functools

import jax
import jax.numpy as jnp
from jax import lax
from jax.experimental import pallas as pl
from jax.experimental.pallas import tpu as pltpu
from jax.experimental.pallas import tpu_sc as plsc

ROWS = 128
N = 32768
K = 64
L = 16
NVREG = N // L
CAP = N + 4 * L
IMIN = -2147483648
NC = 2
NS = 16
ROWS_PER = ROWS // (NC * NS)


def _key(x16):
    u = plsc.bitcast(x16, jnp.int32)
    return u ^ ((u >> 31) & 0x7FFFFFFF)


def _sc_body(x_hbm, o_hbm, xv, ck, ci, wv, wi):
    wid = lax.axis_index("s") * NC + lax.axis_index("c")
    zf = jnp.zeros((L,), jnp.float32)

    def z_body(i, carry):
        for u in range(4):
            ck[pl.ds((i * 4 + u) * L, L)] = zf
        return carry
    lax.fori_loop(0, CAP // (4 * L), z_body, jnp.int32(0))

    def do_row(r, carry):
        row = wid * ROWS_PER + r
        pltpu.sync_copy(x_hbm.at[row], xv)

        def a_body(i, accs):
            accs = list(accs)
            base = i * 8 * L
            for u in range(8):
                x16 = xv[pl.ds(base + u * L, L)]
                accs[u % 4] = jnp.maximum(accs[u % 4], x16)
            return tuple(accs)
        ninf = jnp.full((L,), -jnp.inf, jnp.float32)
        a0, a1, a2, a3 = lax.fori_loop(
            0, NVREG // 8, a_body, (ninf, ninf, ninf, ninf))
        t0 = jnp.min(jnp.minimum(jnp.minimum(a0, a1), jnp.minimum(a2, a3)))

        def b_body(i, off):
            base = i * 4 * L
            xs = [xv[pl.ds(base + u * L, L)] for u in range(4)]
            ms = [x16 >= t0 for x16 in xs]
            kfs = [plsc.bitcast(_key(x16), jnp.float32) for x16 in xs]
            ps = [plsc.all_reduce_population_count(m)[0] for m in ms]
            offs = [off, off + ps[0], off + ps[0] + ps[1],
                    off + ps[0] + ps[1] + ps[2]]
            for u in range(4):
                plsc.store_compressed(ck.at[pl.ds(offs[u], L)], kfs[u],
                                      mask=ms[u])
                i16 = lax.iota(jnp.int32, L) + (base + u * L)
                plsc.store_compressed(ci.at[pl.ds(offs[u], L)], i16,
                                      mask=ms[u])
            return offs[3] + ps[3]
        n = lax.fori_loop(0, NVREG // 4, b_body, jnp.int32(0))

        kpad = plsc.bitcast(jnp.full((L,), IMIN, jnp.int32), jnp.float32)
        for j in range(4):
            ck[pl.ds(n + j * L, L)] = kpad
        nv2 = (n + 2 * L - 1) // (2 * L)

        def count_k(pred):
            def body(j, acc):
                k0 = plsc.bitcast(ck[pl.ds(j * 2 * L, L)], jnp.int32)
                k1 = plsc.bitcast(ck[pl.ds(j * 2 * L + L, L)], jnp.int32)
                return acc + jnp.where(pred(k0), 1, 0) + jnp.where(pred(k1), 1, 0)
            return jnp.sum(lax.fori_loop(0, nv2, body, jnp.zeros((L,), jnp.int32)))

        def count_ki(pred):
            def body(j, acc):
                k0 = plsc.bitcast(ck[pl.ds(j * 2 * L, L)], jnp.int32)
                i0 = ci[pl.ds(j * 2 * L, L)]
                k1 = plsc.bitcast(ck[pl.ds(j * 2 * L + L, L)], jnp.int32)
                i1 = ci[pl.ds(j * 2 * L + L, L)]
                return (acc + jnp.where(pred(k0, i0), 1, 0)
                        + jnp.where(pred(k1, i1), 1, 0))
            return jnp.sum(lax.fori_loop(0, nv2, body, jnp.zeros((L,), jnp.int32)))

        def bit_body(i, w):
            cand_w = w | lax.shift_left(jnp.int32(1), 31 - i)
            cand_v = cand_w ^ IMIN
            cnt = count_k(lambda k16: k16 >= cand_v)
            return jnp.where(cnt >= K, cand_w, w)
        t = lax.fori_loop(0, 32, bit_body, jnp.int32(0)) ^ IMIN

        c_gt = count_k(lambda k16: k16 > t)
        extras = K - c_gt

        def cut_body(i, c):
            cand = c | lax.shift_left(jnp.int32(1), 15 - i)
            h = count_ki(lambda k16, i16: (k16 == t) & (i16 < cand))
            return jnp.where(h <= extras, cand, c)
        cut = lax.fori_loop(0, 16, cut_body, jnp.int32(0))

        def win_body(j, off2):
            for u in range(2):
                k16 = plsc.bitcast(ck[pl.ds(j * 2 * L + u * L, L)], jnp.int32)
                i16 = ci[pl.ds(j * 2 * L + u * L, L)]
                m = (k16 > t) | ((k16 == t) & (i16 < cut))
                val = jnp.maximum(
                    plsc.bitcast(k16 ^ ((k16 >> 31) & 0x7FFFFFFF), jnp.float32),
                    0.0)
                plsc.store_compressed(wv.at[pl.ds(off2, L)], val, mask=m)
                plsc.store_compressed(wi.at[pl.ds(off2, L)], i16, mask=m)
                off2 = off2 + plsc.all_reduce_population_count(m)[0]
            return off2
        lax.fori_loop(0, nv2, win_body, jnp.int32(0))

        nz2 = (n + 4 * L + 2 * L - 1) // (2 * L)
        def rz_body(j, carry):
            ck[pl.ds(j * 2 * L, L)] = zf
            ck[pl.ds(j * 2 * L + L, L)] = zf
            return carry
        lax.fori_loop(0, nz2, rz_body, jnp.int32(0))

        for j in range(4):
            plsc.store_scatter(ck, [wi[pl.ds(j * L, L)]], wv[pl.ds(j * L, L)])
        pltpu.sync_copy(ck.at[pl.ds(0, N)], o_hbm.at[row])
        for j in range(4):
            plsc.store_scatter(ck, [wi[pl.ds(j * L, L)]], zf)
        return carry

    lax.fori_loop(0, ROWS_PER, do_row, jnp.int32(0))


_sc_call = functools.partial(
    pl.kernel,
    out_type=jax.ShapeDtypeStruct((ROWS, N), jnp.float32),
    mesh=plsc.VectorSubcoreMesh(
        core_axis_name="c", subcore_axis_name="s",
        num_cores=NC, num_subcores=NS,
    ),
    scratch_types=[
        pltpu.VMEM((N,), jnp.float32),
        pltpu.VMEM((CAP,), jnp.float32),
        pltpu.VMEM((CAP,), jnp.int32),
        pltpu.VMEM((K + L,), jnp.float32),
        pltpu.VMEM((K + L,), jnp.int32),
    ],
    compiler_params=pltpu.CompilerParams(needs_layout_passes=False),
)(_sc_body)


@jax.jit
def kernel(x, token_mask):
    del token_mask
    return _sc_call(x)


# speedup vs baseline: 11.7228x; 1.1937x over previous
import functools

import jax
import jax.numpy as jnp
from jax import lax
from jax.experimental import pallas as pl
from jax.experimental.pallas import tpu as pltpu
from jax.experimental.pallas import tpu_sc as plsc

ROWS = 128
N = 32768
K = 64
L = 16
NVREG = N // L
CAP = N + 4 * L
IMIN = -2147483648
NC = 2
NS = 16
ROWS_PER = ROWS // (NC * NS)


def _key(x16):
    u = plsc.bitcast(x16, jnp.int32)
    return u ^ ((u >> 31) & 0x7FFFFFFF)


def _sc_body(x_hbm, o_hbm, xv, ck, ci, wv, wi, sem_in, sem_out):
    wid = lax.axis_index("s") * NC + lax.axis_index("c")
    zf = jnp.zeros((L,), jnp.float32)
    row0 = wid * ROWS_PER

    def z_body(i, carry):
        for u in range(4):
            ck[pl.ds((i * 4 + u) * L, L)] = zf
        return carry
    lax.fori_loop(0, CAP // (4 * L), z_body, jnp.int32(0))

    pltpu.sync_copy(x_hbm.at[row0], xv)

    def do_row(r, carry):
        row = row0 + r

        @pl.when(r > 0)
        def _():
            pltpu.make_async_copy(x_hbm.at[row], xv, sem_in).wait()

        def a_body(i, accs):
            accs = list(accs)
            base = i * 8 * L
            for u in range(8):
                x16 = xv[pl.ds(base + u * L, L)]
                accs[u % 4] = jnp.maximum(accs[u % 4], x16)
            return tuple(accs)
        ninf = jnp.full((L,), -jnp.inf, jnp.float32)
        a0, a1, a2, a3 = lax.fori_loop(
            0, NVREG // 8, a_body, (ninf, ninf, ninf, ninf))
        mn4 = jnp.minimum(jnp.minimum(a0, a1), jnp.minimum(a2, a3))
        mx4 = jnp.maximum(jnp.maximum(a0, a1), jnp.maximum(a2, a3))
        t0 = jnp.min(mn4)
        kt0 = jnp.min(_key(mn4))
        kmx = jnp.max(_key(mx4))

        @pl.when(r > 0)
        def _():
            pltpu.make_async_copy(ck.at[pl.ds(0, N)], o_hbm.at[row - 1],
                                  sem_out).wait()
            for j in range(4):
                plsc.store_scatter(ck, [wi[pl.ds(j * L, L)]], zf)

        def b_body(i, off):
            base = i * 4 * L
            xs = [xv[pl.ds(base + u * L, L)] for u in range(4)]
            ms = [x16 >= t0 for x16 in xs]
            kfs = [plsc.bitcast(_key(x16), jnp.float32) for x16 in xs]
            ps = [plsc.all_reduce_population_count(m)[0] for m in ms]
            offs = [off, off + ps[0], off + ps[0] + ps[1],
                    off + ps[0] + ps[1] + ps[2]]
            for u in range(4):
                plsc.store_compressed(ck.at[pl.ds(offs[u], L)], kfs[u],
                                      mask=ms[u])
                i16 = lax.iota(jnp.int32, L) + (base + u * L)
                plsc.store_compressed(ci.at[pl.ds(offs[u], L)], i16,
                                      mask=ms[u])
            return offs[3] + ps[3]
        n = lax.fori_loop(0, NVREG // 4, b_body, jnp.int32(0))

        @pl.when(r < ROWS_PER - 1)
        def _():
            pltpu.async_copy(x_hbm.at[row + 1], xv, sem_in)

        kpad = plsc.bitcast(jnp.full((L,), IMIN, jnp.int32), jnp.float32)
        for j in range(4):
            ck[pl.ds(n + j * L, L)] = kpad
        nv2 = (n + 2 * L - 1) // (2 * L)

        def count_k(pred):
            def body(j, acc):
                k0 = plsc.bitcast(ck[pl.ds(j * 2 * L, L)], jnp.int32)
                k1 = plsc.bitcast(ck[pl.ds(j * 2 * L + L, L)], jnp.int32)
                return acc + jnp.where(pred(k0), 1, 0) + jnp.where(pred(k1), 1, 0)
            return jnp.sum(lax.fori_loop(0, nv2, body, jnp.zeros((L,), jnp.int32)))

        def count_ki(pred):
            def body(j, acc):
                k0 = plsc.bitcast(ck[pl.ds(j * 2 * L, L)], jnp.int32)
                i0 = ci[pl.ds(j * 2 * L, L)]
                k1 = plsc.bitcast(ck[pl.ds(j * 2 * L + L, L)], jnp.int32)
                i1 = ci[pl.ds(j * 2 * L + L, L)]
                return (acc + jnp.where(pred(k0, i0), 1, 0)
                        + jnp.where(pred(k1, i1), 1, 0))
            return jnp.sum(lax.fori_loop(0, nv2, body, jnp.zeros((L,), jnp.int32)))

        pdiff = jnp.broadcast_to(kt0 ^ kmx, (L,))
        ev = (plsc.bitcast(pdiff.astype(jnp.float32), jnp.int32) >> 23) & 0xFF
        nbits = jnp.clip(jnp.max(ev) - 127 + 2, 0, 32)
        nbc = jnp.minimum(nbits, 31)
        wmask = ~(lax.shift_left(jnp.int32(1), nbc) - 1)
        w_init = jnp.where(nbits >= 32, 0, (kmx ^ IMIN) & wmask)

        def bit_body(i, w):
            cand_w = w | lax.shift_left(jnp.int32(1), 31 - i)
            cand_v = cand_w ^ IMIN
            cnt = count_k(lambda k16: k16 >= cand_v)
            return jnp.where(cnt >= K, cand_w, w)
        t = lax.fori_loop(32 - nbits, 32, bit_body, w_init) ^ IMIN

        c_gt = count_k(lambda k16: k16 > t)
        c_eq = count_k(lambda k16: k16 == t)
        extras = K - c_gt

        def cut_search():
            def cut_body(i, c):
                cand = c | lax.shift_left(jnp.int32(1), 15 - i)
                h = count_ki(lambda k16, i16: (k16 == t) & (i16 < cand))
                return jnp.where(h <= extras, cand, c)
            return lax.fori_loop(0, 16, cut_body, jnp.int32(0))
        cut = lax.cond(c_eq == extras, lambda: jnp.int32(N), cut_search)

        def win_body(j, off2):
            for u in range(2):
                k16 = plsc.bitcast(ck[pl.ds(j * 2 * L + u * L, L)], jnp.int32)
                i16 = ci[pl.ds(j * 2 * L + u * L, L)]
                m = (k16 > t) | ((k16 == t) & (i16 < cut))
                val = jnp.maximum(
                    plsc.bitcast(k16 ^ ((k16 >> 31) & 0x7FFFFFFF), jnp.float32),
                    0.0)
                plsc.store_compressed(wv.at[pl.ds(off2, L)], val, mask=m)
                plsc.store_compressed(wi.at[pl.ds(off2, L)], i16, mask=m)
                off2 = off2 + plsc.all_reduce_population_count(m)[0]
            return off2
        lax.fori_loop(0, nv2, win_body, jnp.int32(0))

        nz2 = (n + 4 * L + 2 * L - 1) // (2 * L)
        def rz_body(j, carry2):
            ck[pl.ds(j * 2 * L, L)] = zf
            ck[pl.ds(j * 2 * L + L, L)] = zf
            return carry2
        lax.fori_loop(0, nz2, rz_body, jnp.int32(0))

        for j in range(4):
            plsc.store_scatter(ck, [wi[pl.ds(j * L, L)]], wv[pl.ds(j * L, L)])
        pltpu.async_copy(ck.at[pl.ds(0, N)], o_hbm.at[row], sem_out)
        return carry

    lax.fori_loop(0, ROWS_PER, do_row, jnp.int32(0))
    pltpu.make_async_copy(ck.at[pl.ds(0, N)], o_hbm.at[row0 + ROWS_PER - 1],
                          sem_out).wait()


_sc_call = functools.partial(
    pl.kernel,
    out_type=jax.ShapeDtypeStruct((ROWS, N), jnp.float32),
    mesh=plsc.VectorSubcoreMesh(
        core_axis_name="c", subcore_axis_name="s",
        num_cores=NC, num_subcores=NS,
    ),
    scratch_types=[
        pltpu.VMEM((N,), jnp.float32),
        pltpu.VMEM((CAP,), jnp.float32),
        pltpu.VMEM((CAP,), jnp.int32),
        pltpu.VMEM((K + L,), jnp.float32),
        pltpu.VMEM((K + L,), jnp.int32),
        pltpu.SemaphoreType.DMA,
        pltpu.SemaphoreType.DMA,
    ],
    compiler_params=pltpu.CompilerParams(needs_layout_passes=False),
)(_sc_body)


@jax.jit
def kernel(x, token_mask):
    del token_mask
    return _sc_call(x)


# speedup vs baseline: 14.3120x; 1.2209x over previous
import functools

import jax
import jax.numpy as jnp
from jax import lax
from jax.experimental import pallas as pl
from jax.experimental.pallas import tpu as pltpu
from jax.experimental.pallas import tpu_sc as plsc

ROWS = 128
N = 32768
K = 64
L = 16
NVREG = N // L
CAP = N + 4 * L
IMIN = -2147483648
NC = 2
NS = 16
ROWS_PER = ROWS // (NC * NS)


def _key(x16):
    u = plsc.bitcast(x16, jnp.int32)
    return u ^ ((u >> 31) & 0x7FFFFFFF)


def _sc_body(x_hbm, o_hbm, xv, ck, ci, wv, wi, sem_in, sem_out):
    wid = lax.axis_index("s") * NC + lax.axis_index("c")
    zf = jnp.zeros((L,), jnp.float32)
    row0 = wid * ROWS_PER

    def z_body(i, carry):
        for u in range(4):
            ck[pl.ds((i * 4 + u) * L, L)] = zf
        return carry
    lax.fori_loop(0, CAP // (4 * L), z_body, jnp.int32(0))

    pltpu.sync_copy(x_hbm.at[row0], xv)

    def do_row(r, carry):
        row = row0 + r

        @pl.when(r > 0)
        def _():
            pltpu.make_async_copy(x_hbm.at[row], xv, sem_in).wait()

        def a_body(i, accs):
            accs = list(accs)
            base = i * 16 * L
            for u in range(16):
                x16 = xv[pl.ds(base + u * L, L)]
                accs[u % 4] = jnp.maximum(accs[u % 4], x16)
            return tuple(accs)
        ninf = jnp.full((L,), -jnp.inf, jnp.float32)
        a0, a1, a2, a3 = lax.fori_loop(
            0, NVREG // 16, a_body, (ninf, ninf, ninf, ninf))
        mn4 = jnp.minimum(jnp.minimum(a0, a1), jnp.minimum(a2, a3))
        mx4 = jnp.maximum(jnp.maximum(a0, a1), jnp.maximum(a2, a3))
        t0 = jnp.min(mn4)
        kt0 = jnp.min(_key(mn4))
        kmx = jnp.max(_key(mx4))

        @pl.when(r > 0)
        def _():
            pltpu.make_async_copy(ck.at[pl.ds(0, N)], o_hbm.at[row - 1],
                                  sem_out).wait()
            for j in range(4):
                plsc.store_scatter(ck, [wi[pl.ds(j * L, L)]], zf)

        BU = 8
        def b_body(i, off):
            base = i * BU * L
            xs = [xv[pl.ds(base + u * L, L)] for u in range(BU)]
            ms = [x16 >= t0 for x16 in xs]
            kfs = [plsc.bitcast(_key(x16), jnp.float32) for x16 in xs]
            ps = [plsc.all_reduce_population_count(m)[0] for m in ms]
            offs = [off]
            for u in range(BU - 1):
                offs.append(offs[-1] + ps[u])
            for u in range(BU):
                plsc.store_compressed(ck.at[pl.ds(offs[u], L)], kfs[u],
                                      mask=ms[u])
                i16 = lax.iota(jnp.int32, L) + (base + u * L)
                plsc.store_compressed(ci.at[pl.ds(offs[u], L)], i16,
                                      mask=ms[u])
            return offs[BU - 1] + ps[BU - 1]
        n = lax.fori_loop(0, NVREG // BU, b_body, jnp.int32(0))

        @pl.when(r < ROWS_PER - 1)
        def _():
            pltpu.async_copy(x_hbm.at[row + 1], xv, sem_in)

        kpad = plsc.bitcast(jnp.full((L,), IMIN, jnp.int32), jnp.float32)
        for j in range(4):
            ck[pl.ds(n + j * L, L)] = kpad
        nv2 = (n + 2 * L - 1) // (2 * L)

        def count_k(pred):
            def body(j, acc):
                k0 = plsc.bitcast(ck[pl.ds(j * 2 * L, L)], jnp.int32)
                k1 = plsc.bitcast(ck[pl.ds(j * 2 * L + L, L)], jnp.int32)
                return acc + jnp.where(pred(k0), 1, 0) + jnp.where(pred(k1), 1, 0)
            return jnp.sum(lax.fori_loop(0, nv2, body, jnp.zeros((L,), jnp.int32)))

        def count_ki(pred):
            def body(j, acc):
                k0 = plsc.bitcast(ck[pl.ds(j * 2 * L, L)], jnp.int32)
                i0 = ci[pl.ds(j * 2 * L, L)]
                k1 = plsc.bitcast(ck[pl.ds(j * 2 * L + L, L)], jnp.int32)
                i1 = ci[pl.ds(j * 2 * L + L, L)]
                return (acc + jnp.where(pred(k0, i0), 1, 0)
                        + jnp.where(pred(k1, i1), 1, 0))
            return jnp.sum(lax.fori_loop(0, nv2, body, jnp.zeros((L,), jnp.int32)))

        pdiff = jnp.broadcast_to(kt0 ^ kmx, (L,))
        ev = (plsc.bitcast(pdiff.astype(jnp.float32), jnp.int32) >> 23) & 0xFF
        nbits = jnp.clip(jnp.max(ev) - 127 + 2, 0, 32)
        nbc = jnp.minimum(nbits, 31)
        wmask = ~(lax.shift_left(jnp.int32(1), nbc) - 1)
        w_init = jnp.where(nbits >= 32, 0, (kmx ^ IMIN) & wmask)

        def bit_body(i, w):
            cand_w = w | lax.shift_left(jnp.int32(1), 31 - i)
            cand_v = cand_w ^ IMIN
            cnt = count_k(lambda k16: k16 >= cand_v)
            return jnp.where(cnt >= K, cand_w, w)
        t = lax.fori_loop(32 - nbits, 32, bit_body, w_init) ^ IMIN

        c_gt = count_k(lambda k16: k16 > t)
        c_eq = count_k(lambda k16: k16 == t)
        extras = K - c_gt

        def cut_search():
            def cut_body(i, c):
                cand = c | lax.shift_left(jnp.int32(1), 15 - i)
                h = count_ki(lambda k16, i16: (k16 == t) & (i16 < cand))
                return jnp.where(h <= extras, cand, c)
            return lax.fori_loop(0, 16, cut_body, jnp.int32(0))
        cut = lax.cond(c_eq == extras, lambda: jnp.int32(N), cut_search)

        def win_body(j, off2):
            for u in range(2):
                k16 = plsc.bitcast(ck[pl.ds(j * 2 * L + u * L, L)], jnp.int32)
                i16 = ci[pl.ds(j * 2 * L + u * L, L)]
                m = (k16 > t) | ((k16 == t) & (i16 < cut))
                val = jnp.maximum(
                    plsc.bitcast(k16 ^ ((k16 >> 31) & 0x7FFFFFFF), jnp.float32),
                    0.0)
                plsc.store_compressed(wv.at[pl.ds(off2, L)], val, mask=m)
                plsc.store_compressed(wi.at[pl.ds(off2, L)], i16, mask=m)
                off2 = off2 + plsc.all_reduce_population_count(m)[0]
            return off2
        lax.fori_loop(0, nv2, win_body, jnp.int32(0))

        nz2 = (n + 4 * L + 2 * L - 1) // (2 * L)
        def rz_body(j, carry2):
            ck[pl.ds(j * 2 * L, L)] = zf
            ck[pl.ds(j * 2 * L + L, L)] = zf
            return carry2
        lax.fori_loop(0, nz2, rz_body, jnp.int32(0))

        for j in range(4):
            plsc.store_scatter(ck, [wi[pl.ds(j * L, L)]], wv[pl.ds(j * L, L)])
        pltpu.async_copy(ck.at[pl.ds(0, N)], o_hbm.at[row], sem_out)
        return carry

    lax.fori_loop(0, ROWS_PER, do_row, jnp.int32(0))
    pltpu.make_async_copy(ck.at[pl.ds(0, N)], o_hbm.at[row0 + ROWS_PER - 1],
                          sem_out).wait()


_sc_call = functools.partial(
    pl.kernel,
    out_type=jax.ShapeDtypeStruct((ROWS, N), jnp.float32),
    mesh=plsc.VectorSubcoreMesh(
        core_axis_name="c", subcore_axis_name="s",
        num_cores=NC, num_subcores=NS,
    ),
    scratch_types=[
        pltpu.VMEM((N,), jnp.float32),
        pltpu.VMEM((CAP,), jnp.float32),
        pltpu.VMEM((CAP,), jnp.int32),
        pltpu.VMEM((K + L,), jnp.float32),
        pltpu.VMEM((K + L,), jnp.int32),
        pltpu.SemaphoreType.DMA,
        pltpu.SemaphoreType.DMA,
    ],
    compiler_params=pltpu.CompilerParams(needs_layout_passes=False),
)(_sc_body)


@jax.jit
def kernel(x, token_mask):
    del token_mask
    return _sc_call(x)
